# Initial kernel scaffold; baseline (speedup 1.0000x reference)
import jax
import jax.numpy as jnp
from jax.experimental import pallas as pl


def kernel(pos, vel, idx, val_pos, val_vel):
    raise NotImplementedError("write your pallas kernel here")


# speedup vs baseline: 6.4321x; 6.4321x over previous
import functools

import jax
import jax.numpy as jnp
from jax import lax
from jax.experimental import pallas as pl
from jax.experimental.pallas import tpu as pltpu
from jax.experimental.pallas import tpu_sc as plsc

N = 4194304
NROWS = 2097152
K = 262144
T = 0.5

NC, NS, LANES = 2, 16, 16
NW = NC * NS

_MESH = plsc.VectorSubcoreMesh(
    core_axis_name="c", subcore_axis_name="s", num_cores=NC, num_subcores=NS
)
_SC_PARAMS = pltpu.CompilerParams(needs_layout_passes=False)


def _wid():
    return lax.axis_index("s") * NC + lax.axis_index("c")


SEED_PW = NROWS // NW
CP = 16384
_NCHUNK = 2 * SEED_PW // CP
ZW = 2 * CP


@functools.partial(
    pl.kernel,
    out_type=jax.ShapeDtypeStruct((4 * N,), jnp.float32),
    mesh=_MESH,
    compiler_params=_SC_PARAMS,
    scratch_types=[
        pltpu.VMEM((CP,), jnp.float32),
        pltpu.VMEM((CP,), jnp.float32),
        pltpu.VMEM((ZW,), jnp.float32),
        pltpu.VMEM((ZW,), jnp.float32),
    ],
)
def _dense_pass(pos_hbm, vel_hbm, out_hbm, posb, velb, outb, zb):
    w = _wid()
    lane = lax.iota(jnp.int32, LANES)
    parity = lane & 1
    ppat = 2 * lane - parity
    zero16 = jnp.zeros((LANES,), jnp.float32)

    @pl.loop(0, _NCHUNK)
    def _chunk(c):
        in_base = w * 2 * SEED_PW + c * CP
        pltpu.sync_copy(pos_hbm.at[pl.ds(in_base, CP)], posb)
        pltpu.sync_copy(vel_hbm.at[pl.ds(in_base, CP)], velb)

        @pl.loop(0, CP // LANES)
        def _i(i):
            p = posb[pl.ds(i * LANES, LANES)]
            v = velb[pl.ds(i * LANES, LANES)]
            pat = ppat + 32 * i
            plsc.store_scatter(outb, [pat], p + T * v)
            plsc.store_scatter(outb, [pat + 2], v)

        pltpu.sync_copy(outb, out_hbm.at[pl.ds(2 * in_base, ZW)])

    @pl.loop(0, ZW // LANES)
    def _z(i):
        zb[pl.ds(i * LANES, LANES)] = zero16

    @pl.loop(0, _NCHUNK)
    def _zchunk(c):
        out_base = 4 * (NROWS + w * SEED_PW) + c * ZW
        pltpu.sync_copy(zb, out_hbm.at[pl.ds(out_base, ZW)])


P1C = 16384


@functools.partial(
    pl.kernel,
    out_type=jax.ShapeDtypeStruct((N,), jnp.int32),
    mesh=_MESH,
    compiler_params=_SC_PARAMS,
    scratch_types=[
        pltpu.VMEM((P1C,), jnp.int32),
        pltpu.VMEM((P1C,), jnp.int32),
        pltpu.SemaphoreType.DMA,
    ],
)
def _winner_pass(idx_hbm, jarr_hbm, winner_hbm, idx_v, j_v, sem):
    @pl.when(_wid() == 0)
    def _():
        @pl.loop(0, K // P1C)
        def _chunk(c):
            base = c * P1C
            pltpu.sync_copy(idx_hbm.at[pl.ds(base, P1C)], idx_v)
            pltpu.sync_copy(jarr_hbm.at[pl.ds(base, P1C)], j_v)
            pltpu.async_copy(j_v, winner_hbm.at[idx_v], sem).wait()


JPW = K // NW
C2 = 4096


@functools.partial(
    pl.kernel,
    out_type=(),
    mesh=_MESH,
    compiler_params=_SC_PARAMS,
    scratch_types=[
        pltpu.VMEM((C2,), jnp.int32),
        pltpu.VMEM((C2,), jnp.int32),
        pltpu.VMEM((2 * C2,), jnp.int32),
        pltpu.VMEM((2 * C2,), jnp.float32),
        pltpu.VMEM((2 * C2,), jnp.float32),
        pltpu.VMEM((4 * C2,), jnp.float32),
        pltpu.VMEM((4 * C2,), jnp.int32),
        pltpu.SemaphoreType.DMA,
    ],
)
def _scatter_pass(out_hbm, winner_hbm, idx_hbm, vp_hbm, vv_hbm,
                  idx_v, wv, ig, vp, vv, rows, oi, sem):
    w = _wid()
    lane = lax.iota(jnp.int32, LANES)
    half = lane >> 1
    parity = lane & 1
    quarter = lane >> 2
    rem4 = lane & 3
    ppat = 4 * half + parity

    @pl.loop(0, JPW // C2)
    def _sub(c):
        base = w * JPW + c * C2
        pltpu.sync_copy(idx_hbm.at[pl.ds(base, C2)], idx_v)
        pltpu.async_copy(winner_hbm.at[idx_v], wv, sem).wait()

        @pl.loop(0, 2 * C2 // LANES)
        def _b(k):
            w16 = plsc.load_gather(wv, [half + k * 8])
            ig[pl.ds(k * LANES, LANES)] = w16 * 2 + parity

        pltpu.async_copy(vp_hbm.at[ig], vp, sem).wait()
        pltpu.async_copy(vv_hbm.at[ig], vv, sem).wait()

        @pl.loop(0, 2 * C2 // LANES)
        def _r(k):
            p16 = vp[pl.ds(k * LANES, LANES)]
            v16 = vv[pl.ds(k * LANES, LANES)]
            pat = ppat + 32 * k
            plsc.store_scatter(rows, [pat], p16 + T * v16)
            plsc.store_scatter(rows, [pat + 2], v16)

        @pl.loop(0, 4 * C2 // LANES)
        def _o(k):
            i16 = plsc.load_gather(idx_v, [quarter + k * 4])
            oi[pl.ds(k * LANES, LANES)] = i16 * 4 + rem4

        pltpu.async_copy(rows, out_hbm.at[oi], sem).wait()


def kernel(pos, vel, idx, val_pos, val_vel):
    idx = idx.astype(jnp.int32)
    jarr = jnp.arange(K, dtype=jnp.int32)
    pos_f = pos.reshape(-1)
    vel_f = vel.reshape(-1)
    vp_f = val_pos.reshape(-1)
    vv_f = val_vel.reshape(-1)

    dense = _dense_pass(pos_f, vel_f)
    winner = _winner_pass(idx, jarr)

    out_ref = jax.new_ref(dense)
    _scatter_pass(out_ref, winner, idx, vp_f, vv_f)
    return out_ref[...].reshape(N, 4)


# speedup vs baseline: 39.6795x; 6.1690x over previous
import functools

import jax
import jax.numpy as jnp
from jax import lax
from jax.experimental import pallas as pl
from jax.experimental.pallas import tpu as pltpu
from jax.experimental.pallas import tpu_sc as plsc

N = 4194304
NROWS = 2097152
K = 262144
T = 0.5

NC, NS, LANES = 2, 16, 16
NW = NC * NS

_MESH = plsc.VectorSubcoreMesh(
    core_axis_name="c", subcore_axis_name="s", num_cores=NC, num_subcores=NS
)
_SC_PARAMS = pltpu.CompilerParams(needs_layout_passes=False)


def _wid():
    return lax.axis_index("s") * NC + lax.axis_index("c")


PAN_PW = (NROWS // 128) // NW
CPAN = 64
_NCHUNK = PAN_PW // CPAN
_PIN = 256 * CPAN
_POUT = 512 * CPAN


@functools.partial(
    pl.kernel,
    out_type=jax.ShapeDtypeStruct((4 * N,), jnp.float32),
    mesh=_MESH,
    compiler_params=_SC_PARAMS,
    scratch_types=[
        pltpu.VMEM((_PIN,), jnp.float32),
        pltpu.VMEM((_PIN,), jnp.float32),
        pltpu.VMEM((_POUT,), jnp.float32),
        pltpu.VMEM((_POUT,), jnp.float32),
    ],
)
def _dense_pass(pos_hbm, vel_hbm, out_hbm, posb, velb, outb, zb):
    w = _wid()
    zero16 = jnp.zeros((LANES,), jnp.float32)

    @pl.loop(0, _NCHUNK)
    def _chunk(c):
        in_base = (w * PAN_PW + c * CPAN) * 256
        pltpu.sync_copy(pos_hbm.at[pl.ds(in_base, _PIN)], posb)
        pltpu.sync_copy(vel_hbm.at[pl.ds(in_base, _PIN)], velb)

        @pl.loop(0, CPAN)
        def _q(q):
            @pl.loop(0, 256 // LANES)
            def _t(t):
                src = q * 256 + t * LANES
                p = posb[pl.ds(src, LANES)]
                v = velb[pl.ds(src, LANES)]
                dst = q * 512 + t * LANES
                outb[pl.ds(dst, LANES)] = p + T * v
                outb[pl.ds(dst + 256, LANES)] = v

        pltpu.sync_copy(outb, out_hbm.at[pl.ds(2 * in_base, _POUT)])

    @pl.loop(0, _POUT // LANES)
    def _z(i):
        zb[pl.ds(i * LANES, LANES)] = zero16

    @pl.loop(0, _NCHUNK)
    def _zchunk(c):
        out_base = 4 * NROWS + (w * _NCHUNK + c) * _POUT
        pltpu.sync_copy(zb, out_hbm.at[pl.ds(out_base, _POUT)])


P1C = 16384


@functools.partial(
    pl.kernel,
    out_type=jax.ShapeDtypeStruct((N,), jnp.int32),
    mesh=_MESH,
    compiler_params=_SC_PARAMS,
    scratch_types=[
        pltpu.VMEM((P1C,), jnp.int32),
        pltpu.VMEM((P1C,), jnp.int32),
        pltpu.SemaphoreType.DMA,
    ],
)
def _winner_pass(idx_hbm, jarr_hbm, winner_hbm, idx_v, j_v, sem):
    @pl.when(_wid() == 0)
    def _():
        @pl.loop(0, K // P1C)
        def _chunk(c):
            base = c * P1C
            pltpu.sync_copy(idx_hbm.at[pl.ds(base, P1C)], idx_v)
            pltpu.sync_copy(jarr_hbm.at[pl.ds(base, P1C)], j_v)
            pltpu.async_copy(j_v, winner_hbm.at[idx_v], sem).wait()


JPW = K // NW
C2 = 4096


@functools.partial(
    pl.kernel,
    out_type=(),
    mesh=_MESH,
    compiler_params=_SC_PARAMS,
    scratch_types=[
        pltpu.VMEM((C2,), jnp.int32),
        pltpu.VMEM((C2,), jnp.int32),
        pltpu.VMEM((2 * C2,), jnp.int32),
        pltpu.VMEM((2 * C2,), jnp.float32),
        pltpu.VMEM((2 * C2,), jnp.float32),
        pltpu.VMEM((4 * C2,), jnp.float32),
        pltpu.VMEM((4 * C2,), jnp.int32),
        pltpu.SemaphoreType.DMA,
    ],
)
def _scatter_pass(out_hbm, winner_hbm, idx_hbm, vp_hbm, vv_hbm,
                  idx_v, wv, ig, vp, vv, rows, oi, sem):
    w = _wid()
    lane = lax.iota(jnp.int32, LANES)
    half = lane >> 1
    parity = lane & 1
    quarter = lane >> 2
    rem4 = lane & 3
    ppat = 4 * half + parity

    @pl.loop(0, JPW // C2)
    def _sub(c):
        base = w * JPW + c * C2
        pltpu.sync_copy(idx_hbm.at[pl.ds(base, C2)], idx_v)
        pltpu.async_copy(winner_hbm.at[idx_v], wv, sem).wait()

        @pl.loop(0, 2 * C2 // LANES)
        def _b(k):
            w16 = plsc.load_gather(wv, [half + k * 8])
            ig[pl.ds(k * LANES, LANES)] = (
                (w16 >> 7) * 256 + (w16 & 127) + 128 * parity
            )

        pltpu.async_copy(vp_hbm.at[ig], vp, sem).wait()
        pltpu.async_copy(vv_hbm.at[ig], vv, sem).wait()

        @pl.loop(0, 2 * C2 // LANES)
        def _r(k):
            p16 = vp[pl.ds(k * LANES, LANES)]
            v16 = vv[pl.ds(k * LANES, LANES)]
            pat = ppat + 32 * k
            plsc.store_scatter(rows, [pat], p16 + T * v16)
            plsc.store_scatter(rows, [pat + 2], v16)

        @pl.loop(0, 4 * C2 // LANES)
        def _o(k):
            i16 = plsc.load_gather(idx_v, [quarter + k * 4])
            oi[pl.ds(k * LANES, LANES)] = (
                (i16 >> 7) * 512 + (i16 & 127) + 128 * rem4
            )

        pltpu.async_copy(rows, out_hbm.at[oi], sem).wait()


def _to_panel_flat(x):
    rows, cols = x.shape
    return x.reshape(rows // 128, 128, cols).transpose(0, 2, 1).reshape(-1)


def kernel(pos, vel, idx, val_pos, val_vel):
    idx = idx.astype(jnp.int32)
    jarr = jnp.arange(K, dtype=jnp.int32)
    pos_f = _to_panel_flat(pos)
    vel_f = _to_panel_flat(vel)
    vp_f = _to_panel_flat(val_pos)
    vv_f = _to_panel_flat(val_vel)

    dense = _dense_pass(pos_f, vel_f)
    winner = _winner_pass(idx, jarr)

    out_ref = jax.new_ref(dense)
    _scatter_pass(out_ref, winner, idx, vp_f, vv_f)
    out = out_ref[...]
    return out.reshape(N // 128, 4, 128).transpose(0, 2, 1).reshape(N, 4)


# speedup vs baseline: 40.6600x; 1.0247x over previous
import functools

import jax
import jax.numpy as jnp
from jax import lax
from jax.experimental import pallas as pl
from jax.experimental.pallas import tpu as pltpu
from jax.experimental.pallas import tpu_sc as plsc

N = 4194304
NROWS = 2097152
K = 262144
T = 0.5

NC, NS, LANES = 2, 16, 16
NW = NC * NS

_MESH = plsc.VectorSubcoreMesh(
    core_axis_name="c", subcore_axis_name="s", num_cores=NC, num_subcores=NS
)
_SC_PARAMS = pltpu.CompilerParams(needs_layout_passes=False)


def _wid():
    return lax.axis_index("s") * NC + lax.axis_index("c")


PAN_PW = (NROWS // 128) // NW
CPAN = 32
_NCHUNK = PAN_PW // CPAN
_PIN = 256 * CPAN
_POUT = 512 * CPAN
_NZ = 4 * (N - NROWS) // NW // _POUT


@functools.partial(
    pl.kernel,
    out_type=jax.ShapeDtypeStruct((4 * N,), jnp.float32),
    mesh=_MESH,
    compiler_params=_SC_PARAMS,
    scratch_types=[
        pltpu.VMEM((_PIN,), jnp.float32),
        pltpu.VMEM((_PIN,), jnp.float32),
        pltpu.VMEM((_PIN,), jnp.float32),
        pltpu.VMEM((_PIN,), jnp.float32),
        pltpu.VMEM((_POUT,), jnp.float32),
        pltpu.VMEM((_POUT,), jnp.float32),
        pltpu.VMEM((_POUT,), jnp.float32),
        pltpu.SemaphoreType.DMA,
        pltpu.SemaphoreType.DMA,
        pltpu.SemaphoreType.DMA,
        pltpu.SemaphoreType.DMA,
        pltpu.SemaphoreType.DMA,
    ],
)
def _dense_pass(pos_hbm, vel_hbm, out_hbm,
                posb0, posb1, velb0, velb1, outb0, outb1, zb,
                si0, si1, so0, so1, sz):
    w = _wid()
    zero16 = jnp.zeros((LANES,), jnp.float32)
    pbufs, vbufs, obufs = (posb0, posb1), (velb0, velb1), (outb0, outb1)
    sis, sos = (si0, si1), (so0, so1)

    @pl.loop(0, _POUT // LANES)
    def _z(i):
        zb[pl.ds(i * LANES, LANES)] = zero16

    zdescs = []
    for z in range(_NZ):
        zbase = 4 * NROWS + (w * _NZ + z) * _POUT
        zdescs.append(pltpu.async_copy(zb, out_hbm.at[pl.ds(zbase, _POUT)], sz))

    def _in_base(c):
        return (w * PAN_PW + c * CPAN) * 256

    def _issue_in(c):
        b = c & 1
        return (
            pltpu.async_copy(pos_hbm.at[pl.ds(_in_base(c), _PIN)], pbufs[b], sis[b]),
            pltpu.async_copy(vel_hbm.at[pl.ds(_in_base(c), _PIN)], vbufs[b], sis[b]),
        )

    in_descs = {0: _issue_in(0)}
    out_descs = {}
    for c in range(_NCHUNK):
        b = c & 1
        if c + 1 < _NCHUNK:
            in_descs[c + 1] = _issue_in(c + 1)
        for d in in_descs.pop(c):
            d.wait()
        if c - 2 >= 0:
            out_descs.pop(c - 2).wait()
        pb, vb, ob = pbufs[b], vbufs[b], obufs[b]

        @pl.loop(0, CPAN)
        def _q(q, pb=pb, vb=vb, ob=ob):
            s0 = q * 256
            d0 = q * 512
            for t in range(256 // LANES):
                p = pb[pl.ds(s0 + t * LANES, LANES)]
                v = vb[pl.ds(s0 + t * LANES, LANES)]
                ob[pl.ds(d0 + t * LANES, LANES)] = p + T * v
                ob[pl.ds(d0 + 256 + t * LANES, LANES)] = v

        out_descs[c] = pltpu.async_copy(
            ob, out_hbm.at[pl.ds(2 * _in_base(c), _POUT)], sos[b]
        )
    out_descs.pop(_NCHUNK - 2).wait()
    out_descs.pop(_NCHUNK - 1).wait()
    for d in zdescs:
        d.wait()


P1C = 16384


@functools.partial(
    pl.kernel,
    out_type=jax.ShapeDtypeStruct((N,), jnp.int32),
    mesh=_MESH,
    compiler_params=_SC_PARAMS,
    scratch_types=[
        pltpu.VMEM((P1C,), jnp.int32),
        pltpu.VMEM((P1C,), jnp.int32),
        pltpu.SemaphoreType.DMA,
    ],
)
def _winner_pass(idx_hbm, jarr_hbm, winner_hbm, idx_v, j_v, sem):
    @pl.when(_wid() == 0)
    def _():
        @pl.loop(0, K // P1C)
        def _chunk(c):
            base = c * P1C
            pltpu.sync_copy(idx_hbm.at[pl.ds(base, P1C)], idx_v)
            pltpu.sync_copy(jarr_hbm.at[pl.ds(base, P1C)], j_v)
            pltpu.async_copy(j_v, winner_hbm.at[idx_v], sem).wait()


JPW = K // NW
C2 = 4096


@functools.partial(
    pl.kernel,
    out_type=(),
    mesh=_MESH,
    compiler_params=_SC_PARAMS,
    scratch_types=[
        pltpu.VMEM((C2,), jnp.int32),
        pltpu.VMEM((C2,), jnp.int32),
        pltpu.VMEM((2 * C2,), jnp.int32),
        pltpu.VMEM((2 * C2,), jnp.float32),
        pltpu.VMEM((2 * C2,), jnp.float32),
        pltpu.VMEM((4 * C2,), jnp.float32),
        pltpu.VMEM((4 * C2,), jnp.int32),
        pltpu.SemaphoreType.DMA,
    ],
)
def _scatter_pass(out_hbm, winner_hbm, idx_hbm, vp_hbm, vv_hbm,
                  idx_v, wv, ig, vp, vv, rows, oi, sem):
    w = _wid()
    lane = lax.iota(jnp.int32, LANES)
    half = lane >> 1
    parity = lane & 1
    quarter = lane >> 2
    rem4 = lane & 3
    ppat = 4 * half + parity

    @pl.loop(0, JPW // C2)
    def _sub(c):
        base = w * JPW + c * C2
        pltpu.sync_copy(idx_hbm.at[pl.ds(base, C2)], idx_v)
        pltpu.async_copy(winner_hbm.at[idx_v], wv, sem).wait()

        @pl.loop(0, 2 * C2 // LANES)
        def _b(k):
            w16 = plsc.load_gather(wv, [half + k * 8])
            ig[pl.ds(k * LANES, LANES)] = (
                (w16 >> 7) * 256 + (w16 & 127) + 128 * parity
            )

        pltpu.async_copy(vp_hbm.at[ig], vp, sem).wait()
        pltpu.async_copy(vv_hbm.at[ig], vv, sem).wait()

        @pl.loop(0, 2 * C2 // LANES)
        def _r(k):
            p16 = vp[pl.ds(k * LANES, LANES)]
            v16 = vv[pl.ds(k * LANES, LANES)]
            pat = ppat + 32 * k
            plsc.store_scatter(rows, [pat], p16 + T * v16)
            plsc.store_scatter(rows, [pat + 2], v16)

        @pl.loop(0, 4 * C2 // LANES)
        def _o(k):
            i16 = plsc.load_gather(idx_v, [quarter + k * 4])
            oi[pl.ds(k * LANES, LANES)] = (
                (i16 >> 7) * 512 + (i16 & 127) + 128 * rem4
            )

        pltpu.async_copy(rows, out_hbm.at[oi], sem).wait()


def _to_panel_flat(x):
    rows, cols = x.shape
    return x.reshape(rows // 128, 128, cols).transpose(0, 2, 1).reshape(-1)


def kernel(pos, vel, idx, val_pos, val_vel):
    idx = idx.astype(jnp.int32)
    jarr = jnp.arange(K, dtype=jnp.int32)
    pos_f = _to_panel_flat(pos)
    vel_f = _to_panel_flat(vel)
    vp_f = _to_panel_flat(val_pos)
    vv_f = _to_panel_flat(val_vel)

    dense = _dense_pass(pos_f, vel_f)
    winner = _winner_pass(idx, jarr)

    out_ref = jax.new_ref(dense)
    _scatter_pass(out_ref, winner, idx, vp_f, vv_f)
    out = out_ref[...]
    return out.reshape(N // 128, 4, 128).transpose(0, 2, 1).reshape(N, 4)
